# baseline (speedup 1.0000x reference)
import jax
import jax.numpy as jnp
from jax.experimental import pallas as pl

_B, _N, _F = 4, 2048, 6
_K = 8
_R = 512
_NT = _N // _R
_EPS = 1e-3


def _bn_expr(x, g, b, m, v):
    return g * (x - m) / jnp.sqrt(v + _EPS) + b


def _knn_body(x_ref, xt_ref,
              w1_ref, b1_ref, w2_ref, b2_ref,
              upd1_ref, knn_ref):
    t = pl.program_id(1)
    xb = x_ref[0]
    feat = xb[:, 2:]
    xr = x_ref[0, pl.ds(t * _R, _R), :]
    prx = xr[:, 0:1]
    pry = xr[:, 1:2]
    pcx = xt_ref[0, 0:1, :]
    pcy = xt_ref[0, 1:2, :]
    d = jnp.sqrt((prx - pcx) ** 2 + (pry - pcy) ** 2)
    col = jax.lax.broadcasted_iota(jnp.int32, (_R, _N), 1)
    acc = jnp.zeros((_R, _N), jnp.float32)
    big = jnp.float32(3.0e38)
    for k in range(_K + 1):
        first = jnp.argmin(d, axis=1, keepdims=True)
        oh = col == first
        if k > 0:
            acc = acc + oh.astype(jnp.float32)
            knn_ref[0, :, k - 1] = first[:, 0]
        if k < _K:
            d = jnp.where(oh, big, d)
    agg = jax.lax.dot(acc, feat, preferred_element_type=jnp.float32,
                      precision=jax.lax.Precision.HIGHEST) * 0.125
    h = jnp.maximum(
        jnp.dot(agg, w1_ref[...], preferred_element_type=jnp.float32)
        + b1_ref[...], 0.0)
    upd1_ref[0] = (jnp.dot(h, w2_ref[...], preferred_element_type=jnp.float32)
                   + b2_ref[...])


def _tail_body(xr_ref, upd1_ref, knn_ref,
               g2w1_ref, g2b1_ref, g2w2_ref, g2b2_ref,
               d1w_ref, d1b_ref,
               d2w_ref, d2b_ref, d3w_ref, d3b_ref,
               g2bn_ref, b2bn_ref, m2bn_ref, v2bn_ref, ow_ref, ob_ref,
               out_ref):
    t = pl.program_id(1)
    xr = xr_ref[0]
    pos = xr[:, 0:2]
    table = upd1_ref[0]
    col = jax.lax.broadcasted_iota(jnp.int32, (_R, _N), 1)
    kn = knn_ref[0]
    acc = jnp.zeros((_R, _N), jnp.float32)
    for k in range(_K):
        acc = acc + (col == kn[:, k:k + 1]).astype(jnp.float32)
    agg = jax.lax.dot(acc, table, preferred_element_type=jnp.float32,
                      precision=jax.lax.Precision.HIGHEST) * 0.125
    h = jnp.maximum(
        jnp.dot(agg, g2w1_ref[...], preferred_element_type=jnp.float32)
        + g2b1_ref[...], 0.0)
    upd2 = (jnp.dot(h, g2w2_ref[...], preferred_element_type=jnp.float32)
            + g2b2_ref[...])
    u1r = upd1_ref[0, pl.ds(t * _R, _R), :]
    xcat = jnp.concatenate([pos, u1r, pos, upd2], axis=1)
    y = jnp.maximum(
        jnp.dot(xcat, d1w_ref[...], preferred_element_type=jnp.float32)
        + d1b_ref[...], 0.0)
    y = jnp.maximum(
        jnp.dot(y, d2w_ref[...], preferred_element_type=jnp.float32)
        + d2b_ref[...], 0.0)
    y = jnp.maximum(
        jnp.dot(y, d3w_ref[...], preferred_element_type=jnp.float32)
        + d3b_ref[...], 0.0)
    y = _bn_expr(y, g2bn_ref[...], b2bn_ref[...], m2bn_ref[...], v2bn_ref[...])
    out_ref[0] = (jnp.dot(y, ow_ref[...], preferred_element_type=jnp.float32)
                  + ob_ref[...])


def _full(shape):
    nd = len(shape)
    return pl.BlockSpec(shape, lambda b, t, _n=nd: (0,) * _n)


def kernel(inputs, bn1_gamma, bn1_beta, bn1_mean, bn1_var,
           g1_w1, g1_b1, g1_w2, g1_b2,
           g2_w1, g2_b1, g2_w2, g2_b2,
           d1_w, d1_b, d2_w, d2_b, d3_w, d3_b,
           bn2_gamma, bn2_beta, bn2_mean, bn2_var,
           out_w, out_b):
    f32 = jnp.float32
    xb = _bn_expr(inputs, bn1_gamma, bn1_beta, bn1_mean, bn1_var)
    xbt = jnp.transpose(xb, (0, 2, 1))

    upd1, knn = pl.pallas_call(
        _knn_body,
        grid=(_B, _NT),
        in_specs=[
            pl.BlockSpec((1, _N, _F), lambda b, t: (b, 0, 0)),
            pl.BlockSpec((1, _F, _N), lambda b, t: (b, 0, 0)),
            _full((_F - 2, 32)), _full((1, 32)),
            _full((32, 32)), _full((1, 32)),
        ],
        out_specs=[
            pl.BlockSpec((1, _R, 32), lambda b, t: (b, t, 0)),
            pl.BlockSpec((1, _R, _K), lambda b, t: (b, t, 0)),
        ],
        out_shape=[jax.ShapeDtypeStruct((_B, _N, 32), f32),
                   jax.ShapeDtypeStruct((_B, _N, _K), jnp.int32)],
    )(xb, xbt,
      g1_w1, g1_b1.reshape(1, 32), g1_w2, g1_b2.reshape(1, 32))

    bn2r = [a.reshape(1, 32) for a in (bn2_gamma, bn2_beta, bn2_mean, bn2_var)]

    out = pl.pallas_call(
        _tail_body,
        grid=(_B, _NT),
        in_specs=[
            pl.BlockSpec((1, _R, _F), lambda b, t: (b, t, 0)),
            pl.BlockSpec((1, _N, 32), lambda b, t: (b, 0, 0)),
            pl.BlockSpec((1, _R, _K), lambda b, t: (b, t, 0)),
            _full((32, 32)), _full((1, 32)),
            _full((32, 32)), _full((1, 32)),
            _full((68, 128)), _full((1, 128)),
            _full((128, 64)), _full((1, 64)),
            _full((64, 32)), _full((1, 32)),
            _full((1, 32)), _full((1, 32)), _full((1, 32)), _full((1, 32)),
            _full((32, 1)), _full((1, 1)),
        ],
        out_specs=pl.BlockSpec((1, _R, 1), lambda b, t: (b, t, 0)),
        out_shape=jax.ShapeDtypeStruct((_B, _N, 1), f32),
    )(xb, upd1, knn,
      g2_w1, g2_b1.reshape(1, 32), g2_w2, g2_b2.reshape(1, 32),
      d1_w, d1_b.reshape(1, 128),
      d2_w, d2_b.reshape(1, 64), d3_w, d3_b.reshape(1, 32),
      *bn2r,
      out_w, out_b.reshape(1, 1))
    return out


# speedup vs baseline: 1.1278x; 1.1278x over previous
import functools

import jax
import jax.numpy as jnp
from jax.experimental import pallas as pl
from jax.experimental.pallas import tpu as pltpu
from jax.experimental.pallas import tpu_sc as plsc

_B, _N, _F = 4, 2048, 6
_K = 8
_R = 512
_NT = _N // _R
_EPS = 1e-3

_P = _B * _N
_NW = 32
_PPW = _P // _NW
_G = 16


def _bn_expr(x, g, b, m, v):
    return g * (x - m) / jnp.sqrt(v + _EPS) + b


def _knn_body(x_ref, xt_ref,
              w1_ref, b1_ref, w2_ref, b2_ref,
              upd1_ref, knn_ref):
    t = pl.program_id(1)
    xb = x_ref[0]
    feat = xb[:, 2:]
    xr = x_ref[0, pl.ds(t * _R, _R), :]
    prx = xr[:, 0:1]
    pry = xr[:, 1:2]
    pcx = xt_ref[0, 0:1, :]
    pcy = xt_ref[0, 1:2, :]
    d = jnp.sqrt((prx - pcx) ** 2 + (pry - pcy) ** 2)
    col = jax.lax.broadcasted_iota(jnp.int32, (_R, _N), 1)
    acc = jnp.zeros((_R, _N), jnp.float32)
    big = jnp.float32(3.0e38)
    for k in range(_K + 1):
        first = jnp.argmin(d, axis=1, keepdims=True)
        oh = col == first
        if k > 0:
            acc = acc + oh.astype(jnp.float32)
            knn_ref[0, :, k - 1] = first[:, 0] + pl.program_id(0) * _N
        if k < _K:
            d = jnp.where(oh, big, d)
    agg = jax.lax.dot(acc, feat, preferred_element_type=jnp.float32,
                      precision=jax.lax.Precision.HIGHEST) * 0.125
    h = jnp.maximum(
        jnp.dot(agg, w1_ref[...], preferred_element_type=jnp.float32)
        + b1_ref[...], 0.0)
    upd1_ref[0] = (jnp.dot(h, w2_ref[...], preferred_element_type=jnp.float32)
                   + b2_ref[...])


@functools.partial(
    pl.kernel,
    mesh=plsc.VectorSubcoreMesh(core_axis_name="c", subcore_axis_name="s"),
    out_type=jax.ShapeDtypeStruct((_P, 32), jnp.float32),
    scratch_types=[
        pltpu.VMEM((_PPW * _K,), jnp.int32),
        pltpu.VMEM((2, 128, 128), jnp.float32),
        pltpu.VMEM((_PPW, 32), jnp.float32),
        pltpu.SemaphoreType.DMA,
        pltpu.SemaphoreType.DMA,
    ],
)
def _sc_gather_sum(idx_hbm, table_hbm, out_hbm, idx_v, rows_v, out_v,
                   sem0, sem1):
    wid = jax.lax.axis_index("s") * 2 + jax.lax.axis_index("c")
    base = wid * _PPW
    pltpu.sync_copy(idx_hbm.at[pl.ds(base * _K, _PPW * _K)], idx_v)

    def start(ch, slot, sem):
        off = pl.multiple_of(ch * 128, 128)
        pltpu.async_copy(
            table_hbm.at[idx_v.at[pl.ds(off, 128)]],
            rows_v.at[slot], sem)

    start(0, 0, sem0)
    start(1, 1, sem1)

    def drain(slot, sem):
        pltpu.make_async_copy(table_hbm.at[pl.ds(0, 128)],
                              rows_v.at[slot], sem).wait()

    def accum(ch, slot):
        for pp in range(16):
            r0 = pp * _K
            for c in range(2):
                s = rows_v[slot, r0, pl.ds(c * 16, 16)]
                for k in range(1, _K):
                    s = s + rows_v[slot, r0 + k, pl.ds(c * 16, 16)]
                out_v[ch * 16 + pp, pl.ds(c * 16, 16)] = s

    def body(g2, carry):
        ch = g2 * 2
        drain(0, sem0)
        accum(ch, 0)
        pl.when(g2 < _G // 2 - 1)(lambda: start(ch + 2, 0, sem0))
        drain(1, sem1)
        accum(ch + 1, 1)
        pl.when(g2 < _G // 2 - 1)(lambda: start(ch + 3, 1, sem1))
        return carry

    jax.lax.fori_loop(0, _G // 2, body, 0)
    pltpu.sync_copy(out_v, out_hbm.at[pl.ds(base, _PPW)])


def _tail_body(xr_ref, upd1_ref, aggsum_ref,
               g2w1_ref, g2b1_ref, g2w2_ref, g2b2_ref,
               d1w_ref, d1b_ref,
               d2w_ref, d2b_ref, d3w_ref, d3b_ref,
               g2bn_ref, b2bn_ref, m2bn_ref, v2bn_ref, ow_ref, ob_ref,
               out_ref):
    t = pl.program_id(1)
    xr = xr_ref[0]
    pos = xr[:, 0:2]
    agg = aggsum_ref[0] * 0.125
    h = jnp.maximum(
        jnp.dot(agg, g2w1_ref[...], preferred_element_type=jnp.float32)
        + g2b1_ref[...], 0.0)
    upd2 = (jnp.dot(h, g2w2_ref[...], preferred_element_type=jnp.float32)
            + g2b2_ref[...])
    u1r = upd1_ref[0]
    xcat = jnp.concatenate([pos, u1r, pos, upd2], axis=1)
    y = jnp.maximum(
        jnp.dot(xcat, d1w_ref[...], preferred_element_type=jnp.float32)
        + d1b_ref[...], 0.0)
    y = jnp.maximum(
        jnp.dot(y, d2w_ref[...], preferred_element_type=jnp.float32)
        + d2b_ref[...], 0.0)
    y = jnp.maximum(
        jnp.dot(y, d3w_ref[...], preferred_element_type=jnp.float32)
        + d3b_ref[...], 0.0)
    y = _bn_expr(y, g2bn_ref[...], b2bn_ref[...], m2bn_ref[...], v2bn_ref[...])
    out_ref[0] = (jnp.dot(y, ow_ref[...], preferred_element_type=jnp.float32)
                  + ob_ref[...])


def _full(shape):
    nd = len(shape)
    return pl.BlockSpec(shape, lambda b, t, _n=nd: (0,) * _n)


def kernel(inputs, bn1_gamma, bn1_beta, bn1_mean, bn1_var,
           g1_w1, g1_b1, g1_w2, g1_b2,
           g2_w1, g2_b1, g2_w2, g2_b2,
           d1_w, d1_b, d2_w, d2_b, d3_w, d3_b,
           bn2_gamma, bn2_beta, bn2_mean, bn2_var,
           out_w, out_b):
    f32 = jnp.float32
    xb = _bn_expr(inputs, bn1_gamma, bn1_beta, bn1_mean, bn1_var)
    xbt = jnp.transpose(xb, (0, 2, 1))

    upd1, knn = pl.pallas_call(
        _knn_body,
        grid=(_B, _NT),
        in_specs=[
            pl.BlockSpec((1, _N, _F), lambda b, t: (b, 0, 0)),
            pl.BlockSpec((1, _F, _N), lambda b, t: (b, 0, 0)),
            _full((_F - 2, 32)), _full((1, 32)),
            _full((32, 32)), _full((1, 32)),
        ],
        out_specs=[
            pl.BlockSpec((1, _R, 32), lambda b, t: (b, t, 0)),
            pl.BlockSpec((1, _R, _K), lambda b, t: (b, t, 0)),
        ],
        out_shape=[jax.ShapeDtypeStruct((_B, _N, 32), f32),
                   jax.ShapeDtypeStruct((_B, _N, _K), jnp.int32)],
    )(xb, xbt,
      g1_w1, g1_b1.reshape(1, 32), g1_w2, g1_b2.reshape(1, 32))

    table = jnp.pad(upd1.reshape(_P, 32), ((0, 0), (0, 96)))
    aggsum = _sc_gather_sum(knn.reshape(_P * _K), table)
    aggsum = aggsum.reshape(_B, _N, 32)

    bn2r = [a.reshape(1, 32) for a in (bn2_gamma, bn2_beta, bn2_mean, bn2_var)]

    out = pl.pallas_call(
        _tail_body,
        grid=(_B, _NT),
        in_specs=[
            pl.BlockSpec((1, _R, _F), lambda b, t: (b, t, 0)),
            pl.BlockSpec((1, _R, 32), lambda b, t: (b, t, 0)),
            pl.BlockSpec((1, _R, 32), lambda b, t: (b, t, 0)),
            _full((32, 32)), _full((1, 32)),
            _full((32, 32)), _full((1, 32)),
            _full((68, 128)), _full((1, 128)),
            _full((128, 64)), _full((1, 64)),
            _full((64, 32)), _full((1, 32)),
            _full((1, 32)), _full((1, 32)), _full((1, 32)), _full((1, 32)),
            _full((32, 1)), _full((1, 1)),
        ],
        out_specs=pl.BlockSpec((1, _R, 1), lambda b, t: (b, t, 0)),
        out_shape=jax.ShapeDtypeStruct((_B, _N, 1), f32),
    )(xb, upd1, aggsum,
      g2_w1, g2_b1.reshape(1, 32), g2_w2, g2_b2.reshape(1, 32),
      d1_w, d1_b.reshape(1, 128),
      d2_w, d2_b.reshape(1, 64), d3_w, d3_b.reshape(1, 32),
      *bn2r,
      out_w, out_b.reshape(1, 1))
    return out


# speedup vs baseline: 1.2314x; 1.0919x over previous
import functools

import jax
import jax.numpy as jnp
from jax.experimental import pallas as pl
from jax.experimental.pallas import tpu as pltpu
from jax.experimental.pallas import tpu_sc as plsc

_B, _N, _F = 4, 2048, 6
_K = 8
_R = 512
_NT = _N // _R
_EPS = 1e-3

_P = _B * _N
_NW = 32
_PPW = _P // _NW
_G = 16


def _bn_expr(x, g, b, m, v):
    return g * (x - m) / jnp.sqrt(v + _EPS) + b


def _knn_body(x_ref, xt_ref,
              w1_ref, b1_ref, w2_ref, b2_ref,
              upd1_ref, knn_ref):
    t = pl.program_id(1)
    xb = x_ref[0]
    feat = xb[:, 2:]
    xr = x_ref[0, pl.ds(t * _R, _R), :]
    prx = xr[:, 0:1]
    pry = xr[:, 1:2]
    pcx = xt_ref[0, 0:1, :]
    pcy = xt_ref[0, 1:2, :]
    d = jnp.sqrt((prx - pcx) ** 2 + (pry - pcy) ** 2)
    col = jax.lax.broadcasted_iota(jnp.int32, (_R, _N), 1)
    acc = jnp.zeros((_R, _N), jnp.float32)
    big = jnp.float32(3.0e38)
    for k in range(_K + 1):
        if k == 0:
            first = jnp.min(jnp.where(d == 0.0, col, _N), axis=1,
                            keepdims=True)
        else:
            first = jnp.argmin(d, axis=1, keepdims=True)
        oh = col == first
        if k > 0:
            acc = acc + oh.astype(jnp.float32)
            knn_ref[0, :, k - 1] = first[:, 0] + pl.program_id(0) * _N
        if k < _K:
            d = jnp.where(oh, big, d)
    fh = feat.astype(jnp.bfloat16).astype(jnp.float32)
    r1 = feat - fh
    f1 = r1.astype(jnp.bfloat16).astype(jnp.float32)
    f2 = r1 - f1
    agg = (jax.lax.dot(acc, fh, preferred_element_type=jnp.float32)
           + jax.lax.dot(acc, f1, preferred_element_type=jnp.float32)
           + jax.lax.dot(acc, f2, preferred_element_type=jnp.float32)) * 0.125
    h = jnp.maximum(
        jnp.dot(agg, w1_ref[...], preferred_element_type=jnp.float32)
        + b1_ref[...], 0.0)
    upd1_ref[0] = (jnp.dot(h, w2_ref[...], preferred_element_type=jnp.float32)
                   + b2_ref[...])


@functools.partial(
    pl.kernel,
    mesh=plsc.VectorSubcoreMesh(core_axis_name="c", subcore_axis_name="s"),
    out_type=jax.ShapeDtypeStruct((_P, 32), jnp.float32),
    scratch_types=[
        pltpu.VMEM((_PPW * _K,), jnp.int32),
        pltpu.VMEM((2, 128, 128), jnp.float32),
        pltpu.VMEM((_PPW, 32), jnp.float32),
        pltpu.SemaphoreType.DMA,
        pltpu.SemaphoreType.DMA,
    ],
)
def _sc_gather_sum(idx_hbm, table_hbm, out_hbm, idx_v, rows_v, out_v,
                   sem0, sem1):
    wid = jax.lax.axis_index("s") * 2 + jax.lax.axis_index("c")
    base = wid * _PPW
    pltpu.sync_copy(idx_hbm.at[pl.ds(base * _K, _PPW * _K)], idx_v)

    def start(ch, slot, sem):
        off = pl.multiple_of(ch * 128, 128)
        pltpu.async_copy(
            table_hbm.at[idx_v.at[pl.ds(off, 128)]],
            rows_v.at[slot], sem)

    start(0, 0, sem0)
    start(1, 1, sem1)

    def drain(slot, sem):
        pltpu.make_async_copy(table_hbm.at[pl.ds(0, 128)],
                              rows_v.at[slot], sem).wait()

    def accum(ch, slot):
        for pp in range(16):
            r0 = pp * _K
            for c in range(2):
                s = rows_v[slot, r0, pl.ds(c * 16, 16)]
                for k in range(1, _K):
                    s = s + rows_v[slot, r0 + k, pl.ds(c * 16, 16)]
                out_v[ch * 16 + pp, pl.ds(c * 16, 16)] = s

    def body(g2, carry):
        ch = g2 * 2
        drain(0, sem0)
        accum(ch, 0)
        pl.when(g2 < _G // 2 - 1)(lambda: start(ch + 2, 0, sem0))
        drain(1, sem1)
        accum(ch + 1, 1)
        pl.when(g2 < _G // 2 - 1)(lambda: start(ch + 3, 1, sem1))
        return carry

    jax.lax.fori_loop(0, _G // 2, body, 0)
    pltpu.sync_copy(out_v, out_hbm.at[pl.ds(base, _PPW)])


def _tail_body(xr_ref, upd1_ref, aggsum_ref,
               g2w1_ref, g2b1_ref, g2w2_ref, g2b2_ref,
               d1w_ref, d1b_ref,
               d2w_ref, d2b_ref, d3w_ref, d3b_ref,
               g2bn_ref, b2bn_ref, m2bn_ref, v2bn_ref, ow_ref, ob_ref,
               out_ref):
    t = pl.program_id(1)
    xr = xr_ref[0]
    pos = xr[:, 0:2]
    agg = aggsum_ref[0] * 0.125
    h = jnp.maximum(
        jnp.dot(agg, g2w1_ref[...], preferred_element_type=jnp.float32)
        + g2b1_ref[...], 0.0)
    upd2 = (jnp.dot(h, g2w2_ref[...], preferred_element_type=jnp.float32)
            + g2b2_ref[...])
    u1r = upd1_ref[0]
    xcat = jnp.concatenate([pos, u1r, pos, upd2], axis=1)
    y = jnp.maximum(
        jnp.dot(xcat, d1w_ref[...], preferred_element_type=jnp.float32)
        + d1b_ref[...], 0.0)
    y = jnp.maximum(
        jnp.dot(y, d2w_ref[...], preferred_element_type=jnp.float32)
        + d2b_ref[...], 0.0)
    y = jnp.maximum(
        jnp.dot(y, d3w_ref[...], preferred_element_type=jnp.float32)
        + d3b_ref[...], 0.0)
    y = _bn_expr(y, g2bn_ref[...], b2bn_ref[...], m2bn_ref[...], v2bn_ref[...])
    out_ref[0] = (jnp.dot(y, ow_ref[...], preferred_element_type=jnp.float32)
                  + ob_ref[...])


def _full(shape):
    nd = len(shape)
    return pl.BlockSpec(shape, lambda b, t, _n=nd: (0,) * _n)


def kernel(inputs, bn1_gamma, bn1_beta, bn1_mean, bn1_var,
           g1_w1, g1_b1, g1_w2, g1_b2,
           g2_w1, g2_b1, g2_w2, g2_b2,
           d1_w, d1_b, d2_w, d2_b, d3_w, d3_b,
           bn2_gamma, bn2_beta, bn2_mean, bn2_var,
           out_w, out_b):
    f32 = jnp.float32
    xb = _bn_expr(inputs, bn1_gamma, bn1_beta, bn1_mean, bn1_var)
    xbt = jnp.transpose(xb, (0, 2, 1))

    upd1, knn = pl.pallas_call(
        _knn_body,
        grid=(_B, _NT),
        in_specs=[
            pl.BlockSpec((1, _N, _F), lambda b, t: (b, 0, 0)),
            pl.BlockSpec((1, _F, _N), lambda b, t: (b, 0, 0)),
            _full((_F - 2, 32)), _full((1, 32)),
            _full((32, 32)), _full((1, 32)),
        ],
        out_specs=[
            pl.BlockSpec((1, _R, 32), lambda b, t: (b, t, 0)),
            pl.BlockSpec((1, _R, _K), lambda b, t: (b, t, 0)),
        ],
        out_shape=[jax.ShapeDtypeStruct((_B, _N, 32), f32),
                   jax.ShapeDtypeStruct((_B, _N, _K), jnp.int32)],
    )(xb, xbt,
      g1_w1, g1_b1.reshape(1, 32), g1_w2, g1_b2.reshape(1, 32))

    table = jnp.pad(upd1.reshape(_P, 32), ((0, 0), (0, 96)))
    aggsum = _sc_gather_sum(knn.reshape(_P * _K), table)
    aggsum = aggsum.reshape(_B, _N, 32)

    bn2r = [a.reshape(1, 32) for a in (bn2_gamma, bn2_beta, bn2_mean, bn2_var)]

    out = pl.pallas_call(
        _tail_body,
        grid=(_B, _NT),
        in_specs=[
            pl.BlockSpec((1, _R, _F), lambda b, t: (b, t, 0)),
            pl.BlockSpec((1, _R, 32), lambda b, t: (b, t, 0)),
            pl.BlockSpec((1, _R, 32), lambda b, t: (b, t, 0)),
            _full((32, 32)), _full((1, 32)),
            _full((32, 32)), _full((1, 32)),
            _full((68, 128)), _full((1, 128)),
            _full((128, 64)), _full((1, 64)),
            _full((64, 32)), _full((1, 32)),
            _full((1, 32)), _full((1, 32)), _full((1, 32)), _full((1, 32)),
            _full((32, 1)), _full((1, 1)),
        ],
        out_specs=pl.BlockSpec((1, _R, 1), lambda b, t: (b, t, 0)),
        out_shape=jax.ShapeDtypeStruct((_B, _N, 1), f32),
    )(xb, upd1, aggsum,
      g2_w1, g2_b1.reshape(1, 32), g2_w2, g2_b2.reshape(1, 32),
      d1_w, d1_b.reshape(1, 128),
      d2_w, d2_b.reshape(1, 64), d3_w, d3_b.reshape(1, 32),
      *bn2r,
      out_w, out_b.reshape(1, 1))
    return out


# speedup vs baseline: 1.2406x; 1.0075x over previous
import functools

import jax
import jax.numpy as jnp
from jax.experimental import pallas as pl
from jax.experimental.pallas import tpu as pltpu
from jax.experimental.pallas import tpu_sc as plsc

_B, _N, _F = 4, 2048, 6
_K = 8
_R = 512
_NT = _N // _R
_EPS = 1e-3

_P = _B * _N
_NW = 32
_PPW = _P // _NW
_G = 16


def _bn_expr(x, g, b, m, v):
    return g * (x - m) / jnp.sqrt(v + _EPS) + b


def _knn_body(x_ref, xt_ref,
              w1_ref, b1_ref, w2_ref, b2_ref,
              upd1_ref, knn_ref):
    t = pl.program_id(1)
    xb = x_ref[0]
    feat = xb[:, 2:]
    xr = x_ref[0, pl.ds(t * _R, _R), :]
    prx = xr[:, 0:1]
    pry = xr[:, 1:2]
    pcx = xt_ref[0, 0:1, :]
    pcy = xt_ref[0, 1:2, :]
    d = jnp.sqrt((prx - pcx) ** 2 + (pry - pcy) ** 2)
    col = jax.lax.broadcasted_iota(jnp.int32, (_R, _N), 1)
    acc = jnp.zeros((_R, _N), jnp.float32)
    big = jnp.float32(3.0e38)
    for k in range(_K + 1):
        if k == 0:
            first = jnp.min(jnp.where(d == 0.0, col, _N), axis=1,
                            keepdims=True)
        else:
            first = jnp.argmin(d, axis=1, keepdims=True)
        oh = col == first
        if k > 0:
            acc = acc + oh.astype(jnp.float32)
            knn_ref[0, :, k - 1] = first[:, 0] + pl.program_id(0) * _N
        if k < _K:
            d = jnp.where(oh, big, d)
    fh = feat.astype(jnp.bfloat16).astype(jnp.float32)
    r1 = feat - fh
    f1 = r1.astype(jnp.bfloat16).astype(jnp.float32)
    f2 = r1 - f1
    agg = (jax.lax.dot(acc, fh, preferred_element_type=jnp.float32)
           + jax.lax.dot(acc, f1, preferred_element_type=jnp.float32)
           + jax.lax.dot(acc, f2, preferred_element_type=jnp.float32)) * 0.125
    h = jnp.maximum(
        jnp.dot(agg, w1_ref[...], preferred_element_type=jnp.float32)
        + b1_ref[...], 0.0)
    upd1_ref[0, :, 0:32] = (
        jnp.dot(h, w2_ref[...], preferred_element_type=jnp.float32)
        + b2_ref[...])


@functools.partial(
    pl.kernel,
    mesh=plsc.VectorSubcoreMesh(core_axis_name="c", subcore_axis_name="s"),
    out_type=jax.ShapeDtypeStruct((_P, 32), jnp.float32),
    scratch_types=[
        pltpu.VMEM((_PPW * _K,), jnp.int32),
        pltpu.VMEM((2, 128, 128), jnp.float32),
        pltpu.VMEM((_PPW, 32), jnp.float32),
        pltpu.SemaphoreType.DMA,
        pltpu.SemaphoreType.DMA,
    ],
)
def _sc_gather_sum(idx_hbm, table_hbm, out_hbm, idx_v, rows_v, out_v,
                   sem0, sem1):
    wid = jax.lax.axis_index("s") * 2 + jax.lax.axis_index("c")
    base = wid * _PPW
    pltpu.sync_copy(idx_hbm.at[pl.ds(base * _K, _PPW * _K)], idx_v)

    def start(ch, slot, sem):
        off = pl.multiple_of(ch * 128, 128)
        pltpu.async_copy(
            table_hbm.at[idx_v.at[pl.ds(off, 128)]],
            rows_v.at[slot], sem)

    start(0, 0, sem0)
    start(1, 1, sem1)

    def drain(slot, sem):
        pltpu.make_async_copy(table_hbm.at[pl.ds(0, 128)],
                              rows_v.at[slot], sem).wait()

    def accum(ch, slot):
        for pp in range(16):
            r0 = pp * _K
            for c in range(2):
                s = rows_v[slot, r0, pl.ds(c * 16, 16)]
                for k in range(1, _K):
                    s = s + rows_v[slot, r0 + k, pl.ds(c * 16, 16)]
                out_v[ch * 16 + pp, pl.ds(c * 16, 16)] = s

    def body(g2, carry):
        ch = g2 * 2
        drain(0, sem0)
        accum(ch, 0)
        pl.when(g2 < _G // 2 - 1)(lambda: start(ch + 2, 0, sem0))
        drain(1, sem1)
        accum(ch + 1, 1)
        pl.when(g2 < _G // 2 - 1)(lambda: start(ch + 3, 1, sem1))
        return carry

    jax.lax.fori_loop(0, _G // 2, body, 0)
    pltpu.sync_copy(out_v, out_hbm.at[pl.ds(base, _PPW)])


def _tail_body(xr_ref, upd1_ref, aggsum_ref,
               g2w1_ref, g2b1_ref, g2w2_ref, g2b2_ref,
               d1w_ref, d1b_ref,
               d2w_ref, d2b_ref, d3w_ref, d3b_ref,
               g2bn_ref, b2bn_ref, m2bn_ref, v2bn_ref, ow_ref, ob_ref,
               out_ref):
    t = pl.program_id(1)
    xr = xr_ref[0]
    pos = xr[:, 0:2]
    agg = aggsum_ref[0] * 0.125
    h = jnp.maximum(
        jnp.dot(agg, g2w1_ref[...], preferred_element_type=jnp.float32)
        + g2b1_ref[...], 0.0)
    upd2 = (jnp.dot(h, g2w2_ref[...], preferred_element_type=jnp.float32)
            + g2b2_ref[...])
    u1r = upd1_ref[0][:, 0:32]
    xcat = jnp.concatenate([pos, u1r, pos, upd2], axis=1)
    y = jnp.maximum(
        jnp.dot(xcat, d1w_ref[...], preferred_element_type=jnp.float32)
        + d1b_ref[...], 0.0)
    y = jnp.maximum(
        jnp.dot(y, d2w_ref[...], preferred_element_type=jnp.float32)
        + d2b_ref[...], 0.0)
    y = jnp.maximum(
        jnp.dot(y, d3w_ref[...], preferred_element_type=jnp.float32)
        + d3b_ref[...], 0.0)
    y = _bn_expr(y, g2bn_ref[...], b2bn_ref[...], m2bn_ref[...], v2bn_ref[...])
    out_ref[0] = (jnp.dot(y, ow_ref[...], preferred_element_type=jnp.float32)
                  + ob_ref[...])


def _full(shape):
    nd = len(shape)
    return pl.BlockSpec(shape, lambda b, t, _n=nd: (0,) * _n)


def kernel(inputs, bn1_gamma, bn1_beta, bn1_mean, bn1_var,
           g1_w1, g1_b1, g1_w2, g1_b2,
           g2_w1, g2_b1, g2_w2, g2_b2,
           d1_w, d1_b, d2_w, d2_b, d3_w, d3_b,
           bn2_gamma, bn2_beta, bn2_mean, bn2_var,
           out_w, out_b):
    f32 = jnp.float32
    xb = _bn_expr(inputs, bn1_gamma, bn1_beta, bn1_mean, bn1_var)
    xbt = jnp.transpose(xb, (0, 2, 1))

    upd1, knn = pl.pallas_call(
        _knn_body,
        grid=(_B, _NT),
        in_specs=[
            pl.BlockSpec((1, _N, _F), lambda b, t: (b, 0, 0)),
            pl.BlockSpec((1, _F, _N), lambda b, t: (b, 0, 0)),
            _full((_F - 2, 32)), _full((1, 32)),
            _full((32, 32)), _full((1, 32)),
        ],
        out_specs=[
            pl.BlockSpec((1, _R, 128), lambda b, t: (b, t, 0)),
            pl.BlockSpec((1, _R, _K), lambda b, t: (b, t, 0)),
        ],
        out_shape=[jax.ShapeDtypeStruct((_B, _N, 128), f32),
                   jax.ShapeDtypeStruct((_B, _N, _K), jnp.int32)],
    )(xb, xbt,
      g1_w1, g1_b1.reshape(1, 32), g1_w2, g1_b2.reshape(1, 32))

    aggsum = _sc_gather_sum(knn.reshape(_P * _K), upd1.reshape(_P, 128))
    aggsum = aggsum.reshape(_B, _N, 32)

    bn2r = [a.reshape(1, 32) for a in (bn2_gamma, bn2_beta, bn2_mean, bn2_var)]

    out = pl.pallas_call(
        _tail_body,
        grid=(_B, _NT),
        in_specs=[
            pl.BlockSpec((1, _R, _F), lambda b, t: (b, t, 0)),
            pl.BlockSpec((1, _R, 128), lambda b, t: (b, t, 0)),
            pl.BlockSpec((1, _R, 32), lambda b, t: (b, t, 0)),
            _full((32, 32)), _full((1, 32)),
            _full((32, 32)), _full((1, 32)),
            _full((68, 128)), _full((1, 128)),
            _full((128, 64)), _full((1, 64)),
            _full((64, 32)), _full((1, 32)),
            _full((1, 32)), _full((1, 32)), _full((1, 32)), _full((1, 32)),
            _full((32, 1)), _full((1, 1)),
        ],
        out_specs=pl.BlockSpec((1, _R, 1), lambda b, t: (b, t, 0)),
        out_shape=jax.ShapeDtypeStruct((_B, _N, 1), f32),
    )(xb, upd1, aggsum,
      g2_w1, g2_b1.reshape(1, 32), g2_w2, g2_b2.reshape(1, 32),
      d1_w, d1_b.reshape(1, 128),
      d2_w, d2_b.reshape(1, 64), d3_w, d3_b.reshape(1, 32),
      *bn2r,
      out_w, out_b.reshape(1, 1))
    return out
